# Initial kernel scaffold; baseline (speedup 1.0000x reference)
import functools

import jax
import jax.numpy as jnp
from jax import lax
from jax.experimental import pallas as pl
from jax.experimental.pallas import tpu as pltpu
from jax.experimental.pallas import tpu_sc as plsc

D = 64
K = 20
NC, NS = 2, 16
NW = NC * NS
C = 64
IDXCHUNK = 128


def _sc_scores(center_ids, pos_ids, neg_ids_flat, in_embed, out_embed):
    B = center_ids.shape[0]
    BW = B // NW
    nchunks = BW // C

    mesh = plsc.VectorSubcoreMesh(
        core_axis_name="c", subcore_axis_name="s",
        num_cores=NC, num_subcores=NS)

    @functools.partial(
        pl.kernel,
        out_type=(
            jax.ShapeDtypeStruct((B,), jnp.float32),
            jax.ShapeDtypeStruct((B * K,), jnp.float32),
        ),
        mesh=mesh,
        scratch_types=[
            pltpu.VMEM((C,), jnp.int32),
            pltpu.VMEM((C,), jnp.int32),
            pltpu.VMEM((C * K,), jnp.int32),
            pltpu.VMEM((C, D), jnp.float32),
            pltpu.VMEM((C, D), jnp.float32),
            pltpu.VMEM((C * K, D), jnp.float32),
            pltpu.VMEM((C,), jnp.float32),
            pltpu.VMEM((C * K,), jnp.float32),
            pltpu.SemaphoreType.DMA,
        ],
    )
    def sc_kernel(cids_hbm, pids_hbm, nids_hbm, in_hbm, out_hbm,
                  pos_o, neg_o,
                  cidx, pidx, nidx, cvec, pvec, nvec, ps, ns, sem):
        wid = lax.axis_index("s") * NC + lax.axis_index("c")
        base = wid * BW
        for i in range(nchunks):
            rb = base + i * C
            pltpu.sync_copy(cids_hbm.at[pl.ds(rb, C)], cidx)
            pltpu.sync_copy(pids_hbm.at[pl.ds(rb, C)], pidx)
            pltpu.sync_copy(nids_hbm.at[pl.ds(rb * K, C * K)], nidx)
            pltpu.async_copy(in_hbm.at[cidx], cvec, sem).wait()
            pltpu.async_copy(out_hbm.at[pidx], pvec, sem).wait()
            for j in range(C * K // IDXCHUNK):
                pltpu.async_copy(
                    out_hbm.at[nidx.at[pl.ds(j * IDXCHUNK, IDXCHUNK)]],
                    nvec.at[pl.ds(j * IDXCHUNK, IDXCHUNK)], sem).wait()

            def row(r, carry):
                c = [cvec[r, pl.ds(16 * j, 16)] for j in range(4)]
                p = [pvec[r, pl.ds(16 * j, 16)] for j in range(4)]
                s = c[0] * p[0] + c[1] * p[1] + c[2] * p[2] + c[3] * p[3]
                ps[r] = jnp.sum(s)
                for kk in range(K):
                    n = [nvec[r * K + kk, pl.ds(16 * j, 16)]
                         for j in range(4)]
                    t = c[0] * n[0] + c[1] * n[1] + c[2] * n[2] + c[3] * n[3]
                    ns[r * K + kk] = jnp.sum(t)
                return carry

            lax.fori_loop(0, C, row, 0)
            pltpu.sync_copy(ps, pos_o.at[pl.ds(rb, C)])
            pltpu.sync_copy(ns, neg_o.at[pl.ds(rb * K, C * K)])

    return sc_kernel(center_ids, pos_ids, neg_ids_flat, in_embed, out_embed)


def _tc_loss(pos_s, neg_s, B):
    def body(p_ref, n_ref, o_ref):
        def ls(x):
            return jnp.minimum(x, 0.0) - jnp.log1p(jnp.exp(-jnp.abs(x)))
        tot = jnp.sum(ls(p_ref[...])) + jnp.sum(ls(-n_ref[...]))
        o_ref[0, 0] = -tot / B

    out = pl.pallas_call(
        body,
        out_shape=jax.ShapeDtypeStruct((1, 1), jnp.float32),
        in_specs=[pl.BlockSpec(memory_space=pltpu.VMEM)] * 2,
        out_specs=pl.BlockSpec(memory_space=pltpu.SMEM),
    )(pos_s.reshape(B // 128, 128), neg_s.reshape(B * K // 128, 128))
    return out[0, 0]


def kernel(center_ids, pos_ids, neg_ids, in_embed, out_embed):
    B = center_ids.shape[0]
    pos_s, neg_s = _sc_scores(
        center_ids.astype(jnp.int32),
        pos_ids.astype(jnp.int32),
        neg_ids.reshape(-1).astype(jnp.int32),
        in_embed, out_embed)
    return _tc_loss(pos_s, neg_s, B)


# speedup vs baseline: 3.8701x; 3.8701x over previous
import functools

import jax
import jax.numpy as jnp
from jax import lax
from jax.experimental import pallas as pl
from jax.experimental.pallas import tpu as pltpu
from jax.experimental.pallas import tpu_sc as plsc

D = 64
K = 20
NC, NS = 2, 16
NW = NC * NS
C = 64
IDXCHUNK = 128
L = 16


def _sc_scores(center_ids, pos_ids, neg_ids_flat, in_embed, out_embed):
    B = center_ids.shape[0]
    BW = B // NW
    nchunks = BW // C

    mesh = plsc.VectorSubcoreMesh(
        core_axis_name="c", subcore_axis_name="s",
        num_cores=NC, num_subcores=NS)

    @functools.partial(
        pl.kernel,
        out_type=(
            jax.ShapeDtypeStruct((B,), jnp.float32),
            jax.ShapeDtypeStruct((B * K,), jnp.float32),
        ),
        mesh=mesh,
        compiler_params=pltpu.CompilerParams(
            needs_layout_passes=False, use_tc_tiling_on_sc=False),
        scratch_types=[
            pltpu.VMEM((C,), jnp.int32),
            pltpu.VMEM((C,), jnp.int32),
            pltpu.VMEM((C * K,), jnp.int32),
            pltpu.VMEM((C, D), jnp.float32),
            pltpu.VMEM((C, D), jnp.float32),
            pltpu.VMEM((C * K, D), jnp.float32),
            pltpu.VMEM((C,), jnp.float32),
            pltpu.VMEM((C * K,), jnp.float32),
            pltpu.SemaphoreType.DMA,
        ],
    )
    def sc_kernel(cids_hbm, pids_hbm, nids_hbm, in_hbm, out_hbm,
                  pos_o, neg_o,
                  cidx, pidx, nidx, cvec, pvec, nvec, ps, ns, sem):
        wid = lax.axis_index("s") * NC + lax.axis_index("c")
        base = wid * BW
        lanes = lax.iota(jnp.int32, L)

        def chunk(i, carry0):
            rb = base + i * C
            pltpu.sync_copy(cids_hbm.at[pl.ds(rb, C)], cidx)
            pltpu.sync_copy(pids_hbm.at[pl.ds(rb, C)], pidx)
            pltpu.sync_copy(nids_hbm.at[pl.ds(rb * K, C * K)], nidx)
            pltpu.async_copy(in_hbm.at[cidx], cvec, sem).wait()
            pltpu.async_copy(out_hbm.at[pidx], pvec, sem).wait()
            for j in range(C * K // IDXCHUNK):
                pltpu.async_copy(
                    out_hbm.at[nidx.at[pl.ds(j * IDXCHUNK, IDXCHUNK)]],
                    nvec.at[pl.ds(j * IDXCHUNK, IDXCHUNK)], sem).wait()

            def group(g, carry1):
                rows = g * L + lanes
                rows_k = rows * K

                def dstep(d, accs):
                    dv = jnp.full((L,), 0, jnp.int32) + d
                    cg = plsc.load_gather(cvec, [rows, dv])
                    pg = plsc.load_gather(pvec, [rows, dv])
                    out = [accs[0] + cg * pg]
                    for kk in range(K):
                        ng = plsc.load_gather(nvec, [rows_k + kk, dv])
                        out.append(accs[1 + kk] + ng * cg)
                    return tuple(out)

                zero = jnp.zeros((L,), jnp.float32)
                accs = lax.fori_loop(0, D, dstep, (zero,) * (K + 1))
                ps[pl.ds(g * L, L)] = accs[0]
                for kk in range(K):
                    ns[pl.ds(kk * C + g * L, L)] = accs[1 + kk]
                return carry1

            lax.fori_loop(0, C // L, group, 0)
            pltpu.sync_copy(ps, pos_o.at[pl.ds(rb, C)])
            pltpu.sync_copy(ns, neg_o.at[pl.ds(rb * K, C * K)])
            return carry0

        lax.fori_loop(0, nchunks, chunk, 0)

    return sc_kernel(center_ids, pos_ids, neg_ids_flat, in_embed, out_embed)


def _tc_loss(pos_s, neg_s, B):
    def body(p_ref, n_ref, o_ref):
        def ls(x):
            return jnp.minimum(x, 0.0) - jnp.log1p(jnp.exp(-jnp.abs(x)))
        tot = jnp.sum(ls(p_ref[...])) + jnp.sum(ls(-n_ref[...]))
        o_ref[0, 0] = -tot / B

    out = pl.pallas_call(
        body,
        out_shape=jax.ShapeDtypeStruct((1, 1), jnp.float32),
        in_specs=[pl.BlockSpec(memory_space=pltpu.VMEM)] * 2,
        out_specs=pl.BlockSpec(memory_space=pltpu.SMEM),
    )(pos_s.reshape(B // 128, 128), neg_s.reshape(B * K // 128, 128))
    return out[0, 0]


def kernel(center_ids, pos_ids, neg_ids, in_embed, out_embed):
    B = center_ids.shape[0]
    pos_s, neg_s = _sc_scores(
        center_ids.astype(jnp.int32),
        pos_ids.astype(jnp.int32),
        neg_ids.reshape(-1).astype(jnp.int32),
        in_embed, out_embed)
    return _tc_loss(pos_s, neg_s, B)


# speedup vs baseline: 4.0970x; 1.0586x over previous
import functools

import jax
import jax.numpy as jnp
from jax import lax
from jax.experimental import pallas as pl
from jax.experimental.pallas import tpu as pltpu
from jax.experimental.pallas import tpu_sc as plsc

D = 64
K = 20
NC, NS = 2, 16
NW = NC * NS
C = 32
IDXCHUNK = 128
L = 16


def _sc_scores(center_ids, pos_ids, neg_ids_flat, in_embed, out_embed):
    B = center_ids.shape[0]
    BW = B // NW
    nchunks = BW // C

    mesh = plsc.VectorSubcoreMesh(
        core_axis_name="c", subcore_axis_name="s",
        num_cores=NC, num_subcores=NS)

    @functools.partial(
        pl.kernel,
        out_type=(
            jax.ShapeDtypeStruct((B,), jnp.float32),
            jax.ShapeDtypeStruct((B * K,), jnp.float32),
        ),
        mesh=mesh,
        compiler_params=pltpu.CompilerParams(
            needs_layout_passes=False, use_tc_tiling_on_sc=False),
        scratch_types=[
            pltpu.VMEM((BW,), jnp.int32),
            pltpu.VMEM((BW,), jnp.int32),
            pltpu.VMEM((BW * K,), jnp.int32),
            pltpu.VMEM((2, C, D), jnp.float32),
            pltpu.VMEM((2, C, D), jnp.float32),
            pltpu.VMEM((2, C * K, D), jnp.float32),
            pltpu.VMEM((C,), jnp.float32),
            pltpu.VMEM((C * K,), jnp.float32),
            pltpu.SemaphoreType.DMA,
            pltpu.SemaphoreType.DMA,
        ],
    )
    def sc_kernel(cids_hbm, pids_hbm, nids_hbm, in_hbm, out_hbm,
                  pos_o, neg_o,
                  cidx, pidx, nidx, cvec, pvec, nvec, ps, ns, sem0, sem1):
        wid = lax.axis_index("s") * NC + lax.axis_index("c")
        base = wid * BW
        lanes = lax.iota(jnp.int32, L)
        sems = (sem0, sem1)
        ngath = C * K // IDXCHUNK

        pltpu.sync_copy(cids_hbm.at[pl.ds(base, BW)], cidx)
        pltpu.sync_copy(pids_hbm.at[pl.ds(base, BW)], pidx)
        pltpu.sync_copy(nids_hbm.at[pl.ds(base * K, BW * K)], nidx)

        def fire(i):
            b = i % 2
            sem = sems[b]
            copies = [
                pltpu.async_copy(
                    in_hbm.at[cidx.at[pl.ds(i * C, C)]], cvec.at[b], sem),
                pltpu.async_copy(
                    out_hbm.at[pidx.at[pl.ds(i * C, C)]], pvec.at[b], sem),
            ]
            for j in range(ngath):
                copies.append(pltpu.async_copy(
                    out_hbm.at[nidx.at[pl.ds(i * C * K + j * IDXCHUNK,
                                             IDXCHUNK)]],
                    nvec.at[b, pl.ds(j * IDXCHUNK, IDXCHUNK)], sem))
            return copies

        inflight = fire(0)
        for i in range(nchunks):
            nxt = fire(i + 1) if i + 1 < nchunks else []
            for cp in inflight:
                cp.wait()
            inflight = nxt
            b = i % 2
            rb = base + i * C

            def group(g, carry1):
                rows = g * L + lanes
                rows_k = rows * K

                def dstep(d, accs):
                    dv = jnp.full((L,), 0, jnp.int32) + d
                    cg = plsc.load_gather(cvec.at[b], [rows, dv])
                    pg = plsc.load_gather(pvec.at[b], [rows, dv])
                    out = [accs[0] + cg * pg]
                    for kk in range(K):
                        ng = plsc.load_gather(nvec.at[b], [rows_k + kk, dv])
                        out.append(accs[1 + kk] + ng * cg)
                    return tuple(out)

                zero = jnp.zeros((L,), jnp.float32)
                accs = lax.fori_loop(0, D, dstep, (zero,) * (K + 1))
                ps[pl.ds(g * L, L)] = accs[0]
                for kk in range(K):
                    ns[pl.ds(kk * C + g * L, L)] = accs[1 + kk]
                return carry1

            lax.fori_loop(0, C // L, group, 0)
            pltpu.sync_copy(ps, pos_o.at[pl.ds(rb, C)])
            pltpu.sync_copy(ns, neg_o.at[pl.ds(rb * K, C * K)])

    return sc_kernel(center_ids, pos_ids, neg_ids_flat, in_embed, out_embed)


def _tc_loss(pos_s, neg_s, B):
    def body(p_ref, n_ref, o_ref):
        def ls(x):
            return jnp.minimum(x, 0.0) - jnp.log1p(jnp.exp(-jnp.abs(x)))
        tot = jnp.sum(ls(p_ref[...])) + jnp.sum(ls(-n_ref[...]))
        o_ref[0, 0] = -tot / B

    out = pl.pallas_call(
        body,
        out_shape=jax.ShapeDtypeStruct((1, 1), jnp.float32),
        in_specs=[pl.BlockSpec(memory_space=pltpu.VMEM)] * 2,
        out_specs=pl.BlockSpec(memory_space=pltpu.SMEM),
    )(pos_s.reshape(B // 128, 128), neg_s.reshape(B * K // 128, 128))
    return out[0, 0]


def kernel(center_ids, pos_ids, neg_ids, in_embed, out_embed):
    B = center_ids.shape[0]
    pos_s, neg_s = _sc_scores(
        center_ids.astype(jnp.int32),
        pos_ids.astype(jnp.int32),
        neg_ids.reshape(-1).astype(jnp.int32),
        in_embed, out_embed)
    return _tc_loss(pos_s, neg_s, B)


# speedup vs baseline: 5.4131x; 1.3212x over previous
import functools

import jax
import jax.numpy as jnp
from jax import lax
from jax.experimental import pallas as pl
from jax.experimental.pallas import tpu as pltpu
from jax.experimental.pallas import tpu_sc as plsc

D = 64
K = 20
NC, NS = 2, 16
NW = NC * NS
C = 32
IDXCHUNK = 128
L = 16


def _sc_scores(center_ids, pos_ids, neg_ids_flat, in_embed, out_embed):
    B = center_ids.shape[0]
    BW = B // NW
    nchunks = BW // C

    mesh = plsc.VectorSubcoreMesh(
        core_axis_name="c", subcore_axis_name="s",
        num_cores=NC, num_subcores=NS)

    @functools.partial(
        pl.kernel,
        out_type=(
            jax.ShapeDtypeStruct((B,), jnp.float32),
            jax.ShapeDtypeStruct((B * K,), jnp.float32),
        ),
        mesh=mesh,
        compiler_params=pltpu.CompilerParams(
            needs_layout_passes=False, use_tc_tiling_on_sc=False),
        scratch_types=[
            pltpu.VMEM((BW,), jnp.int32),
            pltpu.VMEM((BW,), jnp.int32),
            pltpu.VMEM((BW * K,), jnp.int32),
            pltpu.VMEM((2, C, D), jnp.float32),
            pltpu.VMEM((2, C, D), jnp.float32),
            pltpu.VMEM((2, C * K, D), jnp.float32),
            pltpu.VMEM((C,), jnp.float32),
            pltpu.VMEM((C * K,), jnp.float32),
            pltpu.SemaphoreType.DMA,
            pltpu.SemaphoreType.DMA,
        ],
    )
    def sc_kernel(cids_hbm, pids_hbm, nids_hbm, in_hbm, out_hbm,
                  pos_o, neg_o,
                  cidx, pidx, nidx, cvec, pvec, nvec, ps, ns, sem0, sem1):
        wid = lax.axis_index("s") * NC + lax.axis_index("c")
        base = wid * BW
        lanes = lax.iota(jnp.int32, L)
        sems = (sem0, sem1)
        ngath = C * K // IDXCHUNK

        pltpu.sync_copy(cids_hbm.at[pl.ds(base, BW)], cidx)
        pltpu.sync_copy(pids_hbm.at[pl.ds(base, BW)], pidx)
        pltpu.sync_copy(nids_hbm.at[pl.ds(base * K, BW * K)], nidx)

        def fire(i):
            b = i % 2
            sem = sems[b]
            copies = [
                pltpu.async_copy(
                    in_hbm.at[cidx.at[pl.ds(i * C, C)]], cvec.at[b], sem),
                pltpu.async_copy(
                    out_hbm.at[pidx.at[pl.ds(i * C, C)]], pvec.at[b], sem),
            ]
            for j in range(ngath):
                copies.append(pltpu.async_copy(
                    out_hbm.at[nidx.at[pl.ds(i * C * K + j * IDXCHUNK,
                                             IDXCHUNK)]],
                    nvec.at[b, pl.ds(j * IDXCHUNK, IDXCHUNK)], sem))
            return copies

        inflight = fire(0)
        for i in range(nchunks):
            nxt = fire(i + 1) if i + 1 < nchunks else []
            for cp in inflight:
                cp.wait()
            inflight = nxt
            b = i % 2
            rb = base + i * C

            def group(g, carry1):
                rows = g * L + lanes
                rows_k = rows * K

                def dstep(d, accs):
                    off = (lanes + d) & (D - 1)
                    cg = plsc.load_gather(cvec.at[b], [rows, off])
                    pg = plsc.load_gather(pvec.at[b], [rows, off])
                    out = [accs[0] + cg * pg]
                    for kk in range(K):
                        ng = plsc.load_gather(nvec.at[b], [rows_k + kk, off])
                        out.append(accs[1 + kk] + ng * cg)
                    return tuple(out)

                zero = jnp.zeros((L,), jnp.float32)
                accs = lax.fori_loop(0, D, dstep, (zero,) * (K + 1))
                ps[pl.ds(g * L, L)] = accs[0]
                for kk in range(K):
                    ns[pl.ds(kk * C + g * L, L)] = accs[1 + kk]
                return carry1

            lax.fori_loop(0, C // L, group, 0)
            pltpu.sync_copy(ps, pos_o.at[pl.ds(rb, C)])
            pltpu.sync_copy(ns, neg_o.at[pl.ds(rb * K, C * K)])

    return sc_kernel(center_ids, pos_ids, neg_ids_flat, in_embed, out_embed)


def _tc_loss(pos_s, neg_s, B):
    def body(p_ref, n_ref, o_ref):
        def ls(x):
            return jnp.minimum(x, 0.0) - jnp.log1p(jnp.exp(-jnp.abs(x)))
        tot = jnp.sum(ls(p_ref[...])) + jnp.sum(ls(-n_ref[...]))
        o_ref[0, 0] = -tot / B

    out = pl.pallas_call(
        body,
        out_shape=jax.ShapeDtypeStruct((1, 1), jnp.float32),
        in_specs=[pl.BlockSpec(memory_space=pltpu.VMEM)] * 2,
        out_specs=pl.BlockSpec(memory_space=pltpu.SMEM),
    )(pos_s.reshape(B // 128, 128), neg_s.reshape(B * K // 128, 128))
    return out[0, 0]


def kernel(center_ids, pos_ids, neg_ids, in_embed, out_embed):
    B = center_ids.shape[0]
    pos_s, neg_s = _sc_scores(
        center_ids.astype(jnp.int32),
        pos_ids.astype(jnp.int32),
        neg_ids.reshape(-1).astype(jnp.int32),
        in_embed, out_embed)
    return _tc_loss(pos_s, neg_s, B)


# speedup vs baseline: 6.3664x; 1.1761x over previous
import functools

import jax
import jax.numpy as jnp
from jax import lax
from jax.experimental import pallas as pl
from jax.experimental.pallas import tpu as pltpu
from jax.experimental.pallas import tpu_sc as plsc

D = 64
DF = 128
K = 20
NC, NS = 2, 16
NW = NC * NS
C = 16
IDXCHUNK = 80
L = 16


def _sc_scores(center_ids, pos_ids, neg_ids_flat, fused):
    B = center_ids.shape[0]
    BW = B // NW
    nchunks = BW // C

    mesh = plsc.VectorSubcoreMesh(
        core_axis_name="c", subcore_axis_name="s",
        num_cores=NC, num_subcores=NS)

    @functools.partial(
        pl.kernel,
        out_type=(
            jax.ShapeDtypeStruct((B,), jnp.float32),
            jax.ShapeDtypeStruct((B * K,), jnp.float32),
        ),
        mesh=mesh,
        compiler_params=pltpu.CompilerParams(needs_layout_passes=False),
        scratch_types=[
            pltpu.VMEM((BW,), jnp.int32),
            pltpu.VMEM((BW,), jnp.int32),
            pltpu.VMEM((BW * K,), jnp.int32),
            pltpu.VMEM((2, C, DF), jnp.float32),
            pltpu.VMEM((2, C, DF), jnp.float32),
            pltpu.VMEM((2, C * K, DF), jnp.float32),
            pltpu.VMEM((C,), jnp.float32),
            pltpu.VMEM((C * K,), jnp.float32),
            pltpu.SemaphoreType.DMA,
            pltpu.SemaphoreType.DMA,
        ],
    )
    def sc_kernel(cids_hbm, pids_hbm, nids_hbm, tab_hbm,
                  pos_o, neg_o,
                  cidx, pidx, nidx, cvec, pvec, nvec, ps, ns, sem0, sem1):
        wid = lax.axis_index("s") * NC + lax.axis_index("c")
        base = wid * BW
        lanes = lax.iota(jnp.int32, L)
        sems = (sem0, sem1)
        ngath = C * K // IDXCHUNK

        pltpu.sync_copy(cids_hbm.at[pl.ds(base, BW)], cidx)
        pltpu.sync_copy(pids_hbm.at[pl.ds(base, BW)], pidx)
        pltpu.sync_copy(nids_hbm.at[pl.ds(base * K, BW * K)], nidx)

        def fire(i):
            b = i % 2
            sem = sems[b]
            copies = [
                pltpu.async_copy(
                    tab_hbm.at[cidx.at[pl.ds(i * C, C)]], cvec.at[b], sem),
                pltpu.async_copy(
                    tab_hbm.at[pidx.at[pl.ds(i * C, C)]], pvec.at[b], sem),
            ]
            for j in range(ngath):
                copies.append(pltpu.async_copy(
                    tab_hbm.at[nidx.at[pl.ds(i * C * K + j * IDXCHUNK,
                                             IDXCHUNK)]],
                    nvec.at[b, pl.ds(j * IDXCHUNK, IDXCHUNK)], sem))
            return copies

        inflight = fire(0)
        for i in range(nchunks):
            nxt = fire(i + 1) if i + 1 < nchunks else []
            for cp in inflight:
                cp.wait()
            inflight = nxt
            b = i % 2
            rb = base + i * C

            def group(g, carry1):
                rows = g * L + lanes
                rows_k = rows * K

                def dstep(d, accs):
                    off = (lanes + d) & (D - 1)
                    cg = plsc.load_gather(cvec.at[b], [rows, off])
                    pg = plsc.load_gather(pvec.at[b], [rows, off + D])
                    out = [accs[0] + cg * pg]
                    for kk in range(K):
                        ng = plsc.load_gather(
                            nvec.at[b], [rows_k + kk, off + D])
                        out.append(accs[1 + kk] + ng * cg)
                    return tuple(out)

                zero = jnp.zeros((L,), jnp.float32)
                accs = lax.fori_loop(0, D, dstep, (zero,) * (K + 1))
                ps[pl.ds(g * L, L)] = accs[0]
                for kk in range(K):
                    ns[pl.ds(kk * C + g * L, L)] = accs[1 + kk]
                return carry1

            lax.fori_loop(0, C // L, group, 0)
            pltpu.sync_copy(ps, pos_o.at[pl.ds(rb, C)])
            pltpu.sync_copy(ns, neg_o.at[pl.ds(rb * K, C * K)])

    return sc_kernel(center_ids, pos_ids, neg_ids_flat, fused)


def _tc_loss(pos_s, neg_s, B):
    def body(p_ref, n_ref, o_ref):
        def ls(x):
            return jnp.minimum(x, 0.0) - jnp.log1p(jnp.exp(-jnp.abs(x)))
        tot = jnp.sum(ls(p_ref[...])) + jnp.sum(ls(-n_ref[...]))
        o_ref[0, 0] = -tot / B

    out = pl.pallas_call(
        body,
        out_shape=jax.ShapeDtypeStruct((1, 1), jnp.float32),
        in_specs=[pl.BlockSpec(memory_space=pltpu.VMEM)] * 2,
        out_specs=pl.BlockSpec(memory_space=pltpu.SMEM),
    )(pos_s.reshape(B // 128, 128), neg_s.reshape(B * K // 128, 128))
    return out[0, 0]


def kernel(center_ids, pos_ids, neg_ids, in_embed, out_embed):
    B = center_ids.shape[0]
    fused = jnp.concatenate([in_embed, out_embed], axis=1)
    pos_s, neg_s = _sc_scores(
        center_ids.astype(jnp.int32),
        pos_ids.astype(jnp.int32),
        neg_ids.reshape(-1).astype(jnp.int32),
        fused)
    return _tc_loss(pos_s, neg_s, B)
